# baseline (speedup 1.0000x reference)
import functools

import jax
import jax.numpy as jnp
import numpy as np
from jax import lax
from jax.experimental import pallas as pl
from jax.experimental.pallas import tpu as pltpu
from jax.experimental.pallas import tpu_sc as plsc

N = 10000
E = 320000
MUL = 16
NUM_RADIAL = 8
HIDDEN = 64
WEIGHT_NUMEL = MUL * MUL

NC = 2
NS = 16
NW = NC * NS

CH = 128
KB = 16
ROWS_PER_TILE = N // NS

NSPLIT = 1
EP = E // NSPLIT

_mesh = plsc.VectorSubcoreMesh(core_axis_name="c", subcore_axis_name="s")
_sc_params = pltpu.CompilerParams(use_tc_tiling_on_sc=False)


def _make_gather(ep):
    nchunk = ep // CH
    trips = -(-nchunk // NW)
    full = (nchunk // NW) // KB

    @functools.partial(
        pl.kernel,
        mesh=_mesh,
        out_type=jax.ShapeDtypeStruct((ep, 128), jnp.float32),
        scratch_types=[
            pltpu.VMEM((KB, CH), jnp.int32),
            pltpu.VMEM((KB, CH, MUL), jnp.float32),
            pltpu.SemaphoreType.DMA,
            pltpu.SemaphoreType.DMA,
            pltpu.SemaphoreType.DMA,
        ],
        compiler_params=_sc_params,
    )
    def gather_k(x_hbm, src_hbm, feat_hbm, idx_v, rows_v, sem_i, sem_g, sem_w):
        wid = lax.axis_index("s") * NC + lax.axis_index("c")

        def body(i, carry):
            js = [wid + (i * KB + b) * NW for b in range(KB)]
            di = [pltpu.async_copy(src_hbm.at[pl.ds(js[b] * CH, CH)],
                                   idx_v.at[b], sem_i) for b in range(KB)]
            for d in di:
                d.wait()
            dg = [pltpu.async_copy(x_hbm.at[idx_v.at[b]], rows_v.at[b], sem_g)
                  for b in range(KB)]
            for d in dg:
                d.wait()
            dw = [pltpu.async_copy(rows_v.at[b],
                                   feat_hbm.at[pl.ds(js[b] * CH, CH),
                                               pl.ds(0, MUL)],
                                   sem_w) for b in range(KB)]
            for d in dw:
                d.wait()
            return carry

        lax.fori_loop(0, full, body, 0)

        def tail(i, carry):
            j = wid + i * NW

            @pl.when(j < nchunk)
            def _():
                base = j * CH
                pltpu.sync_copy(src_hbm.at[pl.ds(base, CH)], idx_v.at[0])
                pltpu.async_copy(x_hbm.at[idx_v.at[0]], rows_v.at[0],
                                 sem_g).wait()
                pltpu.sync_copy(rows_v.at[0],
                                feat_hbm.at[pl.ds(base, CH), pl.ds(0, MUL)])

            return carry

        lax.fori_loop(full * KB, trips, tail, 0)

    return gather_k


def _make_scatter(ep):
    e_half = ep // 2
    nch_core = e_half // CH
    trips = -(-nch_core // NS)
    full = (nch_core // NS) // KB

    @functools.partial(
        pl.kernel,
        mesh=_mesh,
        out_type=jax.ShapeDtypeStruct((2 * N, 128), jnp.float32),
        scratch_types=[
            pltpu.VMEM((KB, CH), jnp.int32),
            pltpu.VMEM((KB, CH, MUL), jnp.float32),
            pltpu.VMEM_SHARED((N, MUL), jnp.float32),
            pltpu.SemaphoreType.DMA,
            pltpu.SemaphoreType.DMA,
            pltpu.SemaphoreType.DMA,
        ],
        compiler_params=_sc_params,
    )
    def scatter_k(m_hbm, dst_hbm, zero_hbm, out_hbm, idx_v, rows_v, acc_sh,
                  sem_i, sem_m, sem_a):
        cid = lax.axis_index("c")
        sid = lax.axis_index("s")
        r0 = sid * ROWS_PER_TILE
        pltpu.sync_copy(zero_hbm.at[pl.ds(r0, ROWS_PER_TILE)],
                        acc_sh.at[pl.ds(r0, ROWS_PER_TILE)])
        plsc.subcore_barrier()

        def body(i, carry):
            js = [sid + (i * KB + b) * NS for b in range(KB)]
            bases = [cid * e_half + js[b] * CH for b in range(KB)]
            di = [pltpu.async_copy(dst_hbm.at[pl.ds(bases[b], CH)],
                                   idx_v.at[b], sem_i) for b in range(KB)]
            dm = [pltpu.async_copy(m_hbm.at[pl.ds(bases[b], CH),
                                            pl.ds(0, MUL)],
                                   rows_v.at[b], sem_m) for b in range(KB)]
            for d in di:
                d.wait()
            for d in dm:
                d.wait()
            da = [pltpu.async_copy(rows_v.at[b], acc_sh.at[idx_v.at[b]],
                                   sem_a, add=True) for b in range(KB)]
            for d in da:
                d.wait()
            return carry

        lax.fori_loop(0, full, body, 0)

        def tail(i, carry):
            j = sid + i * NS

            @pl.when(j < nch_core)
            def _():
                base = cid * e_half + j * CH
                pltpu.sync_copy(dst_hbm.at[pl.ds(base, CH)], idx_v.at[0])
                pltpu.sync_copy(m_hbm.at[pl.ds(base, CH), pl.ds(0, MUL)],
                                rows_v.at[0])
                pltpu.sync_copy(rows_v.at[0], acc_sh.at[idx_v.at[0]], add=True)

            return carry

        lax.fori_loop(full * KB, trips, tail, 0)
        plsc.subcore_barrier()
        pltpu.sync_copy(
            acc_sh.at[pl.ds(r0, ROWS_PER_TILE)],
            out_hbm.at[pl.ds(cid * N + r0, ROWS_PER_TILE), pl.ds(0, MUL)])

    return scatter_k


_gather_k = _make_gather(EP)
_scatter_k = _make_scatter(EP)


_BLK = 8000


def _main_body(feat_ref, el_ref, ea_ref, w1_ref, w2_ref, o_ref):
    feat = feat_ref[...]
    el_t = el_ref[...].reshape(1, _BLK)
    ea_t = ea_ref[...].reshape(1, _BLK)
    xj = feat[:, 0:MUL]
    centers_t = lax.broadcasted_iota(
        jnp.int32, (NUM_RADIAL, 1), 0).astype(jnp.float32) * np.float32(5.0 / 7.0)
    d_t = el_t - centers_t
    radial_t = jnp.exp(-0.5 * d_t * d_t)
    w1 = w1_ref[...] * np.float32(1.0 / np.sqrt(NUM_RADIAL))
    h_t = jnp.dot(w1.T, radial_t, preferred_element_type=jnp.float32)
    h_t = h_t / (1.0 + jnp.exp(-h_t))
    h_t = h_t * (ea_t * np.float32(1.0 / np.sqrt(MUL)))
    h = jnp.transpose(h_t)
    w2 = w2_ref[...] * np.float32(1.0 / np.sqrt(HIDDEN))
    wts = jnp.dot(h, w2, preferred_element_type=jnp.float32)

    u_t = lax.broadcasted_iota(jnp.int32, (MUL, WEIGHT_NUMEL), 0)
    c_t = lax.broadcasted_iota(jnp.int32, (MUL, WEIGHT_NUMEL), 1)
    tile_m = jnp.where(c_t % MUL == u_t, 1.0, 0.0).astype(jnp.float32)
    xt = jnp.dot(xj, tile_m, preferred_element_type=jnp.float32)
    p = wts * xt
    r_s = lax.broadcasted_iota(jnp.int32, (WEIGHT_NUMEL, MUL), 0)
    w_s = lax.broadcasted_iota(jnp.int32, (WEIGHT_NUMEL, MUL), 1)
    seg_m = jnp.where(r_s // MUL == w_s, 1.0, 0.0).astype(jnp.float32)
    m = jnp.dot(p, seg_m, preferred_element_type=jnp.float32)
    o_ref[:, 0:MUL] = m


def _tc_main(feat, el3, ea3, W1, W2):
    ep = feat.shape[0]
    return pl.pallas_call(
        _main_body,
        grid=(ep // _BLK,),
        in_specs=[
            pl.BlockSpec((_BLK, 128), lambda i: (i, 0)),
            pl.BlockSpec((1, 1, _BLK), lambda i: (i, 0, 0)),
            pl.BlockSpec((1, 1, _BLK), lambda i: (i, 0, 0)),
            pl.BlockSpec((NUM_RADIAL, HIDDEN), lambda i: (0, 0)),
            pl.BlockSpec((HIDDEN, WEIGHT_NUMEL), lambda i: (0, 0)),
        ],
        out_specs=pl.BlockSpec((_BLK, 128), lambda i: (i, 0)),
        out_shape=jax.ShapeDtypeStruct((ep, 128), jnp.float32),
    )(feat, el3, ea3, W1, W2)


def _comb_body(pa_ref, x_ref, wsc_ref, o_ref):
    psum = pa_ref[0:N, 0:MUL] + pa_ref[N:2 * N, 0:MUL]
    wsc = wsc_ref[...] * np.float32(1.0 / np.sqrt(MUL))
    sc = jnp.dot(x_ref[...], wsc, preferred_element_type=jnp.float32)
    o_ref[...] = psum + sc


def _tc_combine(pa, x, W_sc):
    return pl.pallas_call(
        _comb_body,
        out_shape=jax.ShapeDtypeStruct((N, MUL), jnp.float32),
    )(pa, x, W_sc)


def kernel(x, edge_attr, edge_length, edge_src, edge_dst, W1, W2, W_sc):
    src = edge_src.astype(jnp.int32)
    dst = edge_dst.astype(jnp.int32)
    zeros = jnp.zeros((N, MUL), dtype=jnp.float32)
    el3 = edge_length.reshape(EP // _BLK, 1, _BLK)
    ea3 = edge_attr.reshape(EP // _BLK, 1, _BLK)
    feat = _gather_k(x, src)
    mfat = _tc_main(feat, el3, ea3, W1, W2)
    pfat = _scatter_k(mfat, dst, zeros)
    return _tc_combine(pfat, x, W_sc)


# speedup vs baseline: 1.0728x; 1.0728x over previous
import functools

import jax
import jax.numpy as jnp
import numpy as np
from jax import lax
from jax.experimental import pallas as pl
from jax.experimental.pallas import tpu as pltpu
from jax.experimental.pallas import tpu_sc as plsc

N = 10000
E = 320000
MUL = 16
NUM_RADIAL = 8
HIDDEN = 64
WEIGHT_NUMEL = MUL * MUL

NC = 2
NS = 16
NW = NC * NS

CH = 128
KB = 8
ROWS_PER_TILE = N // NS

NSPLIT = 2
EP = E // NSPLIT

_mesh = plsc.VectorSubcoreMesh(core_axis_name="c", subcore_axis_name="s")
_sc_params = pltpu.CompilerParams(use_tc_tiling_on_sc=False)


def _make_gather(ep):
    nchunk = ep // CH
    trips = -(-nchunk // NW)
    full = (nchunk // NW) // KB

    @functools.partial(
        pl.kernel,
        mesh=_mesh,
        out_type=jax.ShapeDtypeStruct((ep, 128), jnp.float32),
        scratch_types=[
            pltpu.VMEM((KB, CH), jnp.int32),
            pltpu.VMEM((KB, CH, MUL), jnp.float32),
            pltpu.SemaphoreType.DMA,
            pltpu.SemaphoreType.DMA,
            pltpu.SemaphoreType.DMA,
        ],
        compiler_params=_sc_params,
    )
    def gather_k(x_hbm, src_hbm, feat_hbm, idx_v, rows_v, sem_i, sem_g, sem_w):
        wid = lax.axis_index("s") * NC + lax.axis_index("c")

        def body(i, carry):
            js = [wid + (i * KB + b) * NW for b in range(KB)]
            di = [pltpu.async_copy(src_hbm.at[pl.ds(js[b] * CH, CH)],
                                   idx_v.at[b], sem_i) for b in range(KB)]
            for d in di:
                d.wait()
            dg = [pltpu.async_copy(x_hbm.at[idx_v.at[b]], rows_v.at[b], sem_g)
                  for b in range(KB)]
            for d in dg:
                d.wait()
            dw = [pltpu.async_copy(rows_v.at[b],
                                   feat_hbm.at[pl.ds(js[b] * CH, CH),
                                               pl.ds(0, MUL)],
                                   sem_w) for b in range(KB)]
            for d in dw:
                d.wait()
            return carry

        lax.fori_loop(0, full, body, 0)

        def tail(i, carry):
            j = wid + i * NW

            @pl.when(j < nchunk)
            def _():
                base = j * CH
                pltpu.sync_copy(src_hbm.at[pl.ds(base, CH)], idx_v.at[0])
                pltpu.async_copy(x_hbm.at[idx_v.at[0]], rows_v.at[0],
                                 sem_g).wait()
                pltpu.sync_copy(rows_v.at[0],
                                feat_hbm.at[pl.ds(base, CH), pl.ds(0, MUL)])

            return carry

        lax.fori_loop(full * KB, trips, tail, 0)

    return gather_k


def _make_scatter(ep):
    e_half = ep // 2
    nch_core = e_half // CH
    trips = -(-nch_core // NS)
    full = (nch_core // NS) // KB

    @functools.partial(
        pl.kernel,
        mesh=_mesh,
        out_type=jax.ShapeDtypeStruct((2 * N, 128), jnp.float32),
        scratch_types=[
            pltpu.VMEM((KB, CH), jnp.int32),
            pltpu.VMEM((KB, CH, MUL), jnp.float32),
            pltpu.VMEM_SHARED((N, MUL), jnp.float32),
            pltpu.SemaphoreType.DMA,
            pltpu.SemaphoreType.DMA,
            pltpu.SemaphoreType.DMA,
        ],
        compiler_params=_sc_params,
    )
    def scatter_k(m_hbm, dst_hbm, zero_hbm, out_hbm, idx_v, rows_v, acc_sh,
                  sem_i, sem_m, sem_a):
        cid = lax.axis_index("c")
        sid = lax.axis_index("s")
        r0 = sid * ROWS_PER_TILE
        pltpu.sync_copy(zero_hbm.at[pl.ds(r0, ROWS_PER_TILE)],
                        acc_sh.at[pl.ds(r0, ROWS_PER_TILE)])
        plsc.subcore_barrier()

        def body(i, carry):
            js = [sid + (i * KB + b) * NS for b in range(KB)]
            bases = [cid * e_half + js[b] * CH for b in range(KB)]
            di = [pltpu.async_copy(dst_hbm.at[pl.ds(bases[b], CH)],
                                   idx_v.at[b], sem_i) for b in range(KB)]
            dm = [pltpu.async_copy(m_hbm.at[pl.ds(bases[b], CH),
                                            pl.ds(0, MUL)],
                                   rows_v.at[b], sem_m) for b in range(KB)]
            for d in di:
                d.wait()
            for d in dm:
                d.wait()
            da = [pltpu.async_copy(rows_v.at[b], acc_sh.at[idx_v.at[b]],
                                   sem_a, add=True) for b in range(KB)]
            for d in da:
                d.wait()
            return carry

        lax.fori_loop(0, full, body, 0)

        def tail(i, carry):
            j = sid + i * NS

            @pl.when(j < nch_core)
            def _():
                base = cid * e_half + j * CH
                pltpu.sync_copy(dst_hbm.at[pl.ds(base, CH)], idx_v.at[0])
                pltpu.sync_copy(m_hbm.at[pl.ds(base, CH), pl.ds(0, MUL)],
                                rows_v.at[0])
                pltpu.sync_copy(rows_v.at[0], acc_sh.at[idx_v.at[0]], add=True)

            return carry

        lax.fori_loop(full * KB, trips, tail, 0)
        plsc.subcore_barrier()
        pltpu.sync_copy(
            acc_sh.at[pl.ds(r0, ROWS_PER_TILE)],
            out_hbm.at[pl.ds(cid * N + r0, ROWS_PER_TILE), pl.ds(0, MUL)])

    return scatter_k


_gather_k = _make_gather(EP)
_scatter_k = _make_scatter(EP)


_BLK = 8000


def _main_body(feat_ref, el_ref, ea_ref, w1_ref, w2_ref, o_ref):
    feat = feat_ref[...]
    el_t = el_ref[...].reshape(1, _BLK)
    ea_t = ea_ref[...].reshape(1, _BLK)
    xj = feat[:, 0:MUL]
    centers_t = lax.broadcasted_iota(
        jnp.int32, (NUM_RADIAL, 1), 0).astype(jnp.float32) * np.float32(5.0 / 7.0)
    d_t = el_t - centers_t
    radial_t = jnp.exp(-0.5 * d_t * d_t)
    w1 = w1_ref[...] * np.float32(1.0 / np.sqrt(NUM_RADIAL))
    h_t = jnp.dot(w1.T, radial_t, preferred_element_type=jnp.float32)
    h_t = h_t / (1.0 + jnp.exp(-h_t))
    h_t = h_t * (ea_t * np.float32(1.0 / np.sqrt(MUL)))
    h = jnp.transpose(h_t)
    w2 = w2_ref[...] * np.float32(1.0 / np.sqrt(HIDDEN))
    wts = jnp.dot(h, w2, preferred_element_type=jnp.float32)

    u_t = lax.broadcasted_iota(jnp.int32, (MUL, WEIGHT_NUMEL), 0)
    c_t = lax.broadcasted_iota(jnp.int32, (MUL, WEIGHT_NUMEL), 1)
    tile_m = jnp.where(c_t % MUL == u_t, 1.0, 0.0).astype(jnp.float32)
    xt = jnp.dot(xj, tile_m, preferred_element_type=jnp.float32)
    p = wts * xt
    r_s = lax.broadcasted_iota(jnp.int32, (WEIGHT_NUMEL, MUL), 0)
    w_s = lax.broadcasted_iota(jnp.int32, (WEIGHT_NUMEL, MUL), 1)
    seg_m = jnp.where(r_s // MUL == w_s, 1.0, 0.0).astype(jnp.float32)
    m = jnp.dot(p, seg_m, preferred_element_type=jnp.float32)
    o_ref[:, 0:MUL] = m


def _tc_main(feat, el3, ea3, W1, W2):
    ep = feat.shape[0]
    return pl.pallas_call(
        _main_body,
        grid=(ep // _BLK,),
        in_specs=[
            pl.BlockSpec((_BLK, 128), lambda i: (i, 0)),
            pl.BlockSpec((1, 1, _BLK), lambda i: (i, 0, 0)),
            pl.BlockSpec((1, 1, _BLK), lambda i: (i, 0, 0)),
            pl.BlockSpec((NUM_RADIAL, HIDDEN), lambda i: (0, 0)),
            pl.BlockSpec((HIDDEN, WEIGHT_NUMEL), lambda i: (0, 0)),
        ],
        out_specs=pl.BlockSpec((_BLK, 128), lambda i: (i, 0)),
        out_shape=jax.ShapeDtypeStruct((ep, 128), jnp.float32),
    )(feat, el3, ea3, W1, W2)


def _comb_body(pa_ref, pb_ref, x_ref, wsc_ref, o_ref):
    psum = (pa_ref[0:N, 0:MUL] + pa_ref[N:2 * N, 0:MUL]
            + pb_ref[0:N, 0:MUL] + pb_ref[N:2 * N, 0:MUL])
    wsc = wsc_ref[...] * np.float32(1.0 / np.sqrt(MUL))
    sc = jnp.dot(x_ref[...], wsc, preferred_element_type=jnp.float32)
    o_ref[...] = psum + sc


def _tc_combine(pa, pb, x, W_sc):
    return pl.pallas_call(
        _comb_body,
        out_shape=jax.ShapeDtypeStruct((N, MUL), jnp.float32),
    )(pa, pb, x, W_sc)


def kernel(x, edge_attr, edge_length, edge_src, edge_dst, W1, W2, W_sc):
    src = edge_src.astype(jnp.int32)
    dst = edge_dst.astype(jnp.int32)
    zeros = jnp.zeros((N, MUL), dtype=jnp.float32)
    parts = []
    for s in range(NSPLIT):
        lo, hi = s * EP, (s + 1) * EP
        el3 = edge_length[lo:hi].reshape(EP // _BLK, 1, _BLK)
        ea3 = edge_attr[lo:hi].reshape(EP // _BLK, 1, _BLK)
        feat = _gather_k(x, src[lo:hi])
        mfat = _tc_main(feat, el3, ea3, W1, W2)
        parts.append(_scatter_k(mfat, dst[lo:hi], zeros))
    return _tc_combine(parts[0], parts[1], x, W_sc)


# speedup vs baseline: 1.1157x; 1.0400x over previous
import functools

import jax
import jax.numpy as jnp
import numpy as np
from jax import lax
from jax.experimental import pallas as pl
from jax.experimental.pallas import tpu as pltpu
from jax.experimental.pallas import tpu_sc as plsc

N = 10000
E = 320000
MUL = 16
NUM_RADIAL = 8
HIDDEN = 64
WEIGHT_NUMEL = MUL * MUL

NC = 2
NS = 16
NW = NC * NS

CH = 128
KB = 8
ROWS_PER_TILE = N // NS

NSPLIT = 2
EP = E // NSPLIT

_mesh = plsc.VectorSubcoreMesh(core_axis_name="c", subcore_axis_name="s")
_sc_params = pltpu.CompilerParams(use_tc_tiling_on_sc=False)


def _make_gather(ep):
    nchunk = ep // CH
    trips = -(-nchunk // NW)
    nbatch = -(-trips // KB)

    @functools.partial(
        pl.kernel,
        mesh=_mesh,
        out_type=jax.ShapeDtypeStruct((ep, 128), jnp.float32),
        scratch_types=[
            pltpu.VMEM((KB, CH), jnp.int32),
            pltpu.VMEM((KB, CH, MUL), jnp.float32),
            pltpu.SemaphoreType.DMA,
            pltpu.SemaphoreType.DMA,
            pltpu.SemaphoreType.DMA,
        ],
        compiler_params=_sc_params,
    )
    def gather_k(x_hbm, src_hbm, feat_hbm, idx_v, rows_v, sem_i, sem_g, sem_w):
        wid = lax.axis_index("s") * NC + lax.axis_index("c")

        def body(i, carry):
            js = [jnp.minimum(wid + (i * KB + b) * NW, nchunk - 1)
                  for b in range(KB)]
            di = [pltpu.async_copy(src_hbm.at[pl.ds(js[b] * CH, CH)],
                                   idx_v.at[b], sem_i) for b in range(KB)]
            for d in di:
                d.wait()
            dg = [pltpu.async_copy(x_hbm.at[idx_v.at[b]], rows_v.at[b], sem_g)
                  for b in range(KB)]
            for d in dg:
                d.wait()
            dw = [pltpu.async_copy(rows_v.at[b],
                                   feat_hbm.at[pl.ds(js[b] * CH, CH),
                                               pl.ds(0, MUL)],
                                   sem_w) for b in range(KB)]
            for d in dw:
                d.wait()
            return carry

        lax.fori_loop(0, nbatch, body, 0)

    return gather_k


def _pad_chunks(nch):
    return -(-nch // (NS * KB)) * NS * KB


def _make_scatter(ep):
    e_half = ep // 2
    nch_core = e_half // CH
    nch_pad = _pad_chunks(nch_core)
    nbatch = nch_pad // (NS * KB)

    @functools.partial(
        pl.kernel,
        mesh=_mesh,
        out_type=jax.ShapeDtypeStruct((2 * N, 128), jnp.float32),
        scratch_types=[
            pltpu.VMEM((KB, CH), jnp.int32),
            pltpu.VMEM((KB, CH, MUL), jnp.float32),
            pltpu.VMEM_SHARED((N + 8, MUL), jnp.float32),
            pltpu.SemaphoreType.DMA,
            pltpu.SemaphoreType.DMA,
            pltpu.SemaphoreType.DMA,
        ],
        compiler_params=_sc_params,
    )
    def scatter_k(m_hbm, dstp_hbm, zero_hbm, out_hbm, idx_v, rows_v, acc_sh,
                  sem_i, sem_m, sem_a):
        cid = lax.axis_index("c")
        sid = lax.axis_index("s")
        r0 = sid * ROWS_PER_TILE
        pltpu.sync_copy(zero_hbm.at[pl.ds(r0, ROWS_PER_TILE)],
                        acc_sh.at[pl.ds(r0, ROWS_PER_TILE)])
        plsc.subcore_barrier()

        def body(i, carry):
            js = [sid + (i * KB + b) * NS for b in range(KB)]
            dbases = [cid * (nch_pad * CH) + js[b] * CH for b in range(KB)]
            mbases = [cid * e_half
                      + jnp.minimum(js[b], nch_core - 1) * CH
                      for b in range(KB)]
            di = [pltpu.async_copy(dstp_hbm.at[pl.ds(dbases[b], CH)],
                                   idx_v.at[b], sem_i) for b in range(KB)]
            dm = [pltpu.async_copy(m_hbm.at[pl.ds(mbases[b], CH),
                                            pl.ds(0, MUL)],
                                   rows_v.at[b], sem_m) for b in range(KB)]
            for d in di:
                d.wait()
            for d in dm:
                d.wait()
            da = [pltpu.async_copy(rows_v.at[b], acc_sh.at[idx_v.at[b]],
                                   sem_a, add=True) for b in range(KB)]
            for d in da:
                d.wait()
            return carry

        lax.fori_loop(0, nbatch, body, 0)
        plsc.subcore_barrier()
        pltpu.sync_copy(
            acc_sh.at[pl.ds(r0, ROWS_PER_TILE)],
            out_hbm.at[pl.ds(cid * N + r0, ROWS_PER_TILE), pl.ds(0, MUL)])

    return scatter_k


_gather_k = _make_gather(EP)
_scatter_k = _make_scatter(EP)


_BLK = 8000


def _main_body(feat_ref, el_ref, ea_ref, w1_ref, w2_ref, o_ref):
    feat = feat_ref[...]
    el_t = el_ref[...].reshape(1, _BLK)
    ea_t = ea_ref[...].reshape(1, _BLK)
    xj = feat[:, 0:MUL]
    centers_t = lax.broadcasted_iota(
        jnp.int32, (NUM_RADIAL, 1), 0).astype(jnp.float32) * np.float32(5.0 / 7.0)
    d_t = el_t - centers_t
    radial_t = jnp.exp(-0.5 * d_t * d_t)
    w1 = w1_ref[...] * np.float32(1.0 / np.sqrt(NUM_RADIAL))
    h_t = jnp.dot(w1.T, radial_t, preferred_element_type=jnp.float32)
    h_t = h_t / (1.0 + jnp.exp(-h_t))
    h_t = h_t * (ea_t * np.float32(1.0 / np.sqrt(MUL)))
    h = jnp.transpose(h_t)
    w2 = w2_ref[...] * np.float32(1.0 / np.sqrt(HIDDEN))
    wts = jnp.dot(h, w2, preferred_element_type=jnp.float32)

    u_t = lax.broadcasted_iota(jnp.int32, (MUL, WEIGHT_NUMEL), 0)
    c_t = lax.broadcasted_iota(jnp.int32, (MUL, WEIGHT_NUMEL), 1)
    tile_m = jnp.where(c_t % MUL == u_t, 1.0, 0.0).astype(jnp.float32)
    xt = jnp.dot(xj, tile_m, preferred_element_type=jnp.float32)
    p = wts * xt
    r_s = lax.broadcasted_iota(jnp.int32, (WEIGHT_NUMEL, MUL), 0)
    w_s = lax.broadcasted_iota(jnp.int32, (WEIGHT_NUMEL, MUL), 1)
    seg_m = jnp.where(r_s // MUL == w_s, 1.0, 0.0).astype(jnp.float32)
    m = jnp.dot(p, seg_m, preferred_element_type=jnp.float32)
    o_ref[:, 0:MUL] = m


def _tc_main(feat, el3, ea3, W1, W2):
    ep = feat.shape[0]
    return pl.pallas_call(
        _main_body,
        grid=(ep // _BLK,),
        in_specs=[
            pl.BlockSpec((_BLK, 128), lambda i: (i, 0)),
            pl.BlockSpec((1, 1, _BLK), lambda i: (i, 0, 0)),
            pl.BlockSpec((1, 1, _BLK), lambda i: (i, 0, 0)),
            pl.BlockSpec((NUM_RADIAL, HIDDEN), lambda i: (0, 0)),
            pl.BlockSpec((HIDDEN, WEIGHT_NUMEL), lambda i: (0, 0)),
        ],
        out_specs=pl.BlockSpec((_BLK, 128), lambda i: (i, 0)),
        out_shape=jax.ShapeDtypeStruct((ep, 128), jnp.float32),
    )(feat, el3, ea3, W1, W2)


def _comb_body(pa_ref, pb_ref, x_ref, wsc_ref, o_ref):
    psum = (pa_ref[0:N, 0:MUL] + pa_ref[N:2 * N, 0:MUL]
            + pb_ref[0:N, 0:MUL] + pb_ref[N:2 * N, 0:MUL])
    wsc = wsc_ref[...] * np.float32(1.0 / np.sqrt(MUL))
    sc = jnp.dot(x_ref[...], wsc, preferred_element_type=jnp.float32)
    o_ref[...] = psum + sc


def _tc_combine(pa, pb, x, W_sc):
    return pl.pallas_call(
        _comb_body,
        out_shape=jax.ShapeDtypeStruct((N, MUL), jnp.float32),
    )(pa, pb, x, W_sc)


def kernel(x, edge_attr, edge_length, edge_src, edge_dst, W1, W2, W_sc):
    src = edge_src.astype(jnp.int32)
    dst = edge_dst.astype(jnp.int32)
    zeros = jnp.zeros((N, MUL), dtype=jnp.float32)
    eh = EP // 2
    padc = _pad_chunks(eh // CH) * CH - eh
    fill = jnp.full((padc,), N, dtype=jnp.int32)
    parts = []
    for s in range(NSPLIT):
        lo, hi = s * EP, (s + 1) * EP
        el3 = edge_length[lo:hi].reshape(EP // _BLK, 1, _BLK)
        ea3 = edge_attr[lo:hi].reshape(EP // _BLK, 1, _BLK)
        feat = _gather_k(x, src[lo:hi])
        mfat = _tc_main(feat, el3, ea3, W1, W2)
        ds_ = dst[lo:hi]
        dstp = jnp.concatenate([ds_[:eh], fill, ds_[eh:], fill])
        parts.append(_scatter_k(mfat, dstp, zeros))
    return _tc_combine(parts[0], parts[1], x, W_sc)
